# baseline (speedup 1.0000x reference)
import dataclasses

import jax
import jax.numpy as jnp
from jax import lax
from jax.experimental import pallas as pl
from jax.experimental.pallas import tpu as pltpu
from jax.experimental.pallas import tpu_sc as plsc

_NUM_CORES = 2
_NUM_SUBCORES = 16
_NUM_WORKERS = _NUM_CORES * _NUM_SUBCORES
_CHUNK = 800
_LANES = 16


def kernel(ids, table):
    batch, hist = ids.shape
    vocab, d = table.shape
    num_indices = batch * hist
    per_worker = num_indices // _NUM_WORKERS
    flat = ids.reshape(num_indices)
    table2 = table.reshape(vocab // 2, 2 * d)

    mesh = plsc.VectorSubcoreMesh(core_axis_name="c", subcore_axis_name="s")
    cp = dataclasses.replace(
        pltpu.CompilerParams(),
        use_tc_tiling_on_sc=False,
        needs_layout_passes=False,
    )

    @pl.kernel(
        out_type=jax.ShapeDtypeStruct((batch, hist, d), table.dtype),
        mesh=mesh,
        scratch_types=[
            pltpu.VMEM((_CHUNK,), jnp.int32),
            pltpu.VMEM((_CHUNK,), jnp.int32),
            pltpu.VMEM((_CHUNK, 2 * d), table.dtype),
            pltpu.SemaphoreType.DMA,
        ],
        compiler_params=cp,
    )
    def gather_kernel(table_hbm, ids_hbm, out_hbm, idx_v, q_v, rows_v, sem):
        wid = lax.axis_index("s") * _NUM_CORES + lax.axis_index("c")
        base = wid * per_worker
        b_base = wid * (per_worker // hist)
        nb = _CHUNK // hist
        iota = lax.iota(jnp.int32, _LANES)

        @pl.loop(0, per_worker, step=_CHUNK)
        def _(off):
            pltpu.sync_copy(ids_hbm.at[pl.ds(base + off, _CHUNK)], idx_v)

            @pl.loop(0, _CHUNK, step=_LANES)
            def _(i):
                q_v.at[pl.ds(i, _LANES)][...] = (
                    idx_v.at[pl.ds(i, _LANES)][...] >> 1
                )

            pltpu.async_copy(table_hbm.at[q_v], rows_v, sem).wait()

            @pl.loop(0, _CHUNK)
            def _(k):
                k16 = jnp.full((_LANES,), k, jnp.int32)
                par = plsc.load_gather(idx_v, [k16]) & 1
                for g in range(d // _LANES):
                    cols = par * d + g * _LANES + iota
                    v = plsc.load_gather(rows_v, [k16, cols])
                    rows_v.at[k, pl.ds(g * _LANES, _LANES)][...] = v

            for b in range(nb):
                pltpu.sync_copy(rows_v.at[pl.ds(b * hist, hist), pl.ds(0, d)],
                                out_hbm.at[b_base + off // hist + b])

    return gather_kernel(table2, flat)


# speedup vs baseline: 1.2518x; 1.2518x over previous
import dataclasses

import jax
import jax.numpy as jnp
from jax import lax
from jax.experimental import pallas as pl
from jax.experimental.pallas import tpu as pltpu
from jax.experimental.pallas import tpu_sc as plsc

_NUM_CORES = 2
_NUM_SUBCORES = 16
_NUM_WORKERS = _NUM_CORES * _NUM_SUBCORES
_CHUNK = 1600


def kernel(ids, table):
    batch, hist = ids.shape
    vocab, d = table.shape
    num_indices = batch * hist
    per_worker = num_indices // _NUM_WORKERS
    flat = ids.reshape(num_indices)

    mesh = plsc.VectorSubcoreMesh(core_axis_name="c", subcore_axis_name="s")
    cp = dataclasses.replace(pltpu.CompilerParams(), use_tc_tiling_on_sc=False)

    @pl.kernel(
        out_type=jax.ShapeDtypeStruct((batch, hist, d), table.dtype),
        mesh=mesh,
        scratch_types=[
            pltpu.VMEM((_CHUNK,), jnp.int32),
            pltpu.VMEM((_CHUNK, d), table.dtype),
            pltpu.SemaphoreType.DMA,
        ],
        compiler_params=cp,
    )
    def gather_kernel(table_hbm, ids_hbm, out_hbm, idx_v, rows_v, sem):
        wid = lax.axis_index("s") * _NUM_CORES + lax.axis_index("c")
        base = wid * per_worker
        b_base = wid * (per_worker // hist)
        nb = _CHUNK // hist

        @pl.loop(0, per_worker, step=_CHUNK)
        def _(off):
            pltpu.sync_copy(ids_hbm.at[pl.ds(base + off, _CHUNK)], idx_v)
            pltpu.async_copy(table_hbm.at[idx_v], rows_v, sem).wait()
            for b in range(nb):
                pltpu.sync_copy(rows_v.at[pl.ds(b * hist, hist), :],
                                out_hbm.at[b_base + off // hist + b])

    return gather_kernel(table, flat)
